# baseline (speedup 1.0000x reference)
import functools

import jax
import jax.numpy as jnp
from jax import lax
from jax.experimental import pallas as pl
from jax.experimental.pallas import tpu as pltpu
from jax.experimental.pallas import tpu_sc as plsc

_NUM_AGENTS = 4096
_FEAT = 3
_TOTAL = _NUM_AGENTS * _FEAT


def _body(table_hbm, out_hbm):
    pltpu.sync_copy(table_hbm.at[pl.ds(0, _TOTAL)], out_hbm)


_sc = functools.partial(
    pl.kernel,
    out_type=jax.ShapeDtypeStruct((_TOTAL,), jnp.float32),
    mesh=plsc.ScalarSubcoreMesh(axis_name="c", num_cores=1),
)(_body)


def kernel(pos_phi, num_agents):
    flat = jnp.reshape(pos_phi, (-1,))
    out = _sc(flat)
    return jnp.reshape(out, (_NUM_AGENTS, _FEAT))
